# baseline (speedup 1.0000x reference)
import functools

import jax
import jax.numpy as jnp
from jax import lax
from jax.experimental import pallas as pl
from jax.experimental.pallas import tpu as pltpu
from jax.experimental.pallas import tpu_sc as plsc

NUM_CORES = 2
NUM_SUBCORES = 16
LANES = 16
NW = NUM_CORES * NUM_SUBCORES

B = 4096
D = 64
BPW = B // NW
GROUPS = BPW // LANES


def _svd_body(uids_hbm, iids_hbm, uemb_hbm, iemb_hbm, ubias_hbm, ibias_hbm,
              out_hbm, uid_v, iid_v, urows_v, irows_v, ub_v, ib_v, score_v,
              sem):
    wid = lax.axis_index("s") * NUM_CORES + lax.axis_index("c")
    base = wid * BPW

    pltpu.sync_copy(uids_hbm.at[pl.ds(base, BPW)], uid_v)
    pltpu.sync_copy(iids_hbm.at[pl.ds(base, BPW)], iid_v)

    cps = [
        pltpu.async_copy(uemb_hbm.at[uid_v], urows_v, sem),
        pltpu.async_copy(iemb_hbm.at[iid_v], irows_v, sem),
        pltpu.async_copy(ubias_hbm.at[uid_v], ub_v, sem),
        pltpu.async_copy(ibias_hbm.at[iid_v], ib_v, sem),
    ]
    for cp in cps:
        cp.wait()

    lane = jnp.arange(LANES, dtype=jnp.int32)

    def group(g, _):
        row0 = g * LANES
        rows = row0 + lane
        acc = ub_v[pl.ds(row0, LANES)] + ib_v[pl.ds(row0, LANES)]
        for d in range(D):
            col = jnp.full((LANES,), d, dtype=jnp.int32)
            u = plsc.load_gather(urows_v, [rows, col])
            i = plsc.load_gather(irows_v, [rows, col])
            acc = acc + u * i
        score_v[pl.ds(row0, LANES)] = acc
        return 0

    lax.fori_loop(0, GROUPS, group, 0)

    pltpu.sync_copy(score_v, out_hbm.at[pl.ds(base, BPW)])


@jax.jit
def _svd_score(user_ids, item_ids, user_emb, item_emb, user_bias, item_bias):
    mesh = plsc.VectorSubcoreMesh(core_axis_name="c", subcore_axis_name="s")
    run = functools.partial(
        pl.kernel,
        out_type=jax.ShapeDtypeStruct((B,), jnp.float32),
        mesh=mesh,
        compiler_params=pltpu.CompilerParams(
            needs_layout_passes=False, use_tc_tiling_on_sc=False),
        scratch_types=[
            pltpu.VMEM((BPW,), jnp.int32),
            pltpu.VMEM((BPW,), jnp.int32),
            pltpu.VMEM((BPW, D), jnp.float32),
            pltpu.VMEM((BPW, D), jnp.float32),
            pltpu.VMEM((BPW,), jnp.float32),
            pltpu.VMEM((BPW,), jnp.float32),
            pltpu.VMEM((BPW,), jnp.float32),
            pltpu.SemaphoreType.DMA,
        ],
    )(_svd_body)
    return run(user_ids, item_ids, user_emb, item_emb, user_bias, item_bias)


def kernel(user_ids, item_ids, user_emb, item_emb, user_bias, item_bias,
           average_score):
    del average_score
    score = _svd_score(user_ids, item_ids, user_emb, item_emb,
                       user_bias.reshape(-1), item_bias.reshape(-1))
    return score.reshape(B, 1)
